# baseline (speedup 1.0000x reference)
import jax
import jax.numpy as jnp
import numpy as np
from jax import lax
from jax.experimental import pallas as pl
from jax.experimental.pallas import tpu as pltpu
from jax.experimental.pallas import tpu_sc as plsc

N = 100000
E = 1600000
D = 32
H = 32
G = 16
B = 128
T = 5

NPAD = 100352
EPAD = 1638400
EROWS = EPAD // 128
NPT = NPAD // 16
ERT = EROWS // 16
DRT = EROWS // 32

NP4 = NPAD // 4
BLK4 = 256
NBLK = NP4 // BLK4
NROW4 = N // 4

_mesh = plsc.VectorSubcoreMesh(
    core_axis_name="c", subcore_axis_name="s", num_cores=2, num_subcores=16)


def _deg_body(dstr_hbm, zinit_hbm, onesrow_hbm, out_hbm, idx_v, ones_v, acc_sh, sem):
    c = lax.axis_index("c")
    s = lax.axis_index("s")
    w = c * 16 + s
    nb = s * NPT
    pltpu.sync_copy(zinit_hbm.at[pl.ds(nb, NPT)], acc_sh.at[pl.ds(nb, NPT)])
    pltpu.sync_copy(onesrow_hbm, ones_v)
    plsc.subcore_barrier()

    def outer(i, _):
        pltpu.sync_copy(dstr_hbm.at[pl.ds(w * DRT + i * 16, 16)], idx_v)
        sd = [pltpu.async_copy(ones_v, acc_sh.at[idx_v.at[j]], sem, add=True)
              for j in range(16)]
        for d in sd:
            d.wait()
        return 0

    lax.fori_loop(0, DRT // 16, outer, 0)
    plsc.subcore_barrier()
    pltpu.sync_copy(acc_sh.at[pl.ds(nb, NPT)],
                    out_hbm.at[pl.ds(nb, NPT), c])


_deg_call = pl.kernel(
    _deg_body,
    out_type=jax.ShapeDtypeStruct((NPAD, 2, 16), jnp.float32),
    mesh=_mesh,
    scratch_types=[
        pltpu.VMEM((16, 128), jnp.int32),
        pltpu.VMEM((128, 16), jnp.float32),
        pltpu.VMEM_SHARED((NPAD, 16), jnp.float32),
        pltpu.SemaphoreType.DMA,
    ],
    compiler_params=pltpu.CompilerParams(use_tc_tiling_on_sc=False),
)


def _edge_body(hsg_hbm, zinit_hbm, srcr_hbm, dstr_hbm, out_hbm, src_v, dst_v,
               rows_v, acc_sh, *sems):
    gsems, ssems = sems[:8], sems[8:]
    c = lax.axis_index("c")
    s = lax.axis_index("s")
    nb = s * NPT
    pltpu.sync_copy(zinit_hbm.at[pl.ds(nb, NPT)], acc_sh.at[pl.ds(nb, NPT)])
    plsc.subcore_barrier()
    er0 = s * ERT
    src_c = srcr_hbm
    hs_c = hsg_hbm.at[c]

    def outer(i, _):
        pltpu.sync_copy(src_c.at[pl.ds(er0 + i * 32, 32)], src_v)
        pltpu.sync_copy(dstr_hbm.at[pl.ds(er0 + i * 32, 32)], dst_v)

        def sub(m, _):
            @pl.when(i * 4 + m > 0)
            def _():
                for k in range(8):
                    pltpu.make_async_copy(
                        rows_v.at[k], acc_sh.at[dst_v.at[m * 8 + k]],
                        ssems[k]).wait()

            gd = [pltpu.async_copy(hs_c.at[src_v.at[m * 8 + k]],
                                   rows_v.at[k], gsems[k])
                  for k in range(8)]
            for k in range(8):
                gd[k].wait()
                pltpu.async_copy(rows_v.at[k], acc_sh.at[dst_v.at[m * 8 + k]],
                                 ssems[k], add=True)
            return 0

        lax.fori_loop(0, 4, sub, 0)
        return 0

    lax.fori_loop(0, ERT // 32, outer, 0)
    for k in range(8):
        pltpu.make_async_copy(rows_v.at[k], acc_sh.at[dst_v.at[31]],
                              ssems[k]).wait()
    plsc.subcore_barrier()
    pltpu.sync_copy(acc_sh.at[pl.ds(nb, NPT)],
                    out_hbm.at[pl.ds(nb, NPT), c])


_edge_call = pl.kernel(
    _edge_body,
    out_type=jax.ShapeDtypeStruct((NPAD, 2, 16), jnp.float32),
    mesh=_mesh,
    scratch_types=[
        pltpu.VMEM((32, 128), jnp.int32),
        pltpu.VMEM((32, 128), jnp.int32),
        pltpu.VMEM((8, 128, 16), jnp.float32),
        pltpu.VMEM_SHARED((NPAD, 16), jnp.float32),
    ] + [pltpu.SemaphoreType.DMA] * 16,
    compiler_params=pltpu.CompilerParams(use_tc_tiling_on_sc=False),
)


def _a_body(x_ref, cnt_ref, fc_ref, w14_ref, hs_ref, dq_ref):
    cnt = jnp.dot(jnp.maximum(cnt_ref[...], 0.0), fc_ref[...],
                  preferred_element_type=jnp.float32)
    dq = lax.rsqrt(1.0 + cnt)
    hs_ref[...] = jnp.dot(x_ref[...], w14_ref[...],
                          preferred_element_type=jnp.float32) * dq
    dq_ref[...] = dq


def _tc_a(xp, cntp, Fc, W14):
    return pl.pallas_call(
        _a_body,
        grid=(NBLK,),
        in_specs=[
            pl.BlockSpec((BLK4, 128), lambda i: (i, 0)),
            pl.BlockSpec((BLK4, 128), lambda i: (i, 0)),
            pl.BlockSpec((128, 128), lambda i: (0, 0)),
            pl.BlockSpec((128, 128), lambda i: (0, 0)),
        ],
        out_specs=[
            pl.BlockSpec((BLK4, 128), lambda i: (i, 0)),
            pl.BlockSpec((BLK4, 128), lambda i: (i, 0)),
        ],
        out_shape=[
            jax.ShapeDtypeStruct((NP4, 128), jnp.float32),
            jax.ShapeDtypeStruct((NP4, 128), jnp.float32),
        ],
    )(xp, cntp, Fc, W14)


def _stats_body(agg_ref, hs_ref, dq_ref, b_ref, f_ref, out_ref):
    i = pl.program_id(0)
    t = (agg_ref[...] + hs_ref[...]) * dq_ref[...] + b_ref[...]
    rows = lax.broadcasted_iota(jnp.int32, (BLK4, 1), 0) + i * BLK4
    t = jnp.where(rows < NROW4, t, 0.0)
    s0 = jnp.sum(t, axis=0)
    s1 = jnp.sum(t * t, axis=0)
    stacked = jnp.concatenate(
        [s0[None, :], s1[None, :], jnp.zeros((6, 128), jnp.float32)], axis=0)

    @pl.when(i == 0)
    def _():
        out_ref[...] = jnp.zeros((8, 128), jnp.float32)

    out_ref[...] += jnp.dot(stacked, f_ref[...],
                            preferred_element_type=jnp.float32)


def _tc_stats(agg, hs, dq, b4, F):
    return pl.pallas_call(
        _stats_body,
        grid=(NBLK,),
        in_specs=[
            pl.BlockSpec((BLK4, 128), lambda i: (i, 0)),
            pl.BlockSpec((BLK4, 128), lambda i: (i, 0)),
            pl.BlockSpec((BLK4, 128), lambda i: (i, 0)),
            pl.BlockSpec((1, 128), lambda i: (0, 0)),
            pl.BlockSpec((128, 128), lambda i: (0, 0)),
        ],
        out_specs=pl.BlockSpec((8, 128), lambda i: (0, 0)),
        out_shape=jax.ShapeDtypeStruct((8, 128), jnp.float32),
    )(agg, hs, dq, b4, F)


def _apply_body(agg_ref, hs_ref, dq_ref, st_ref, b_ref, g_ref, be_ref, w_ref,
                out_ref):
    dq = dq_ref[...]
    t = (agg_ref[...] + hs_ref[...]) * dq + b_ref[...]
    m = st_ref[0:1, :] / N
    v = st_ref[1:2, :] / N - m * m
    sc = g_ref[...] * lax.rsqrt(v + 1e-5)
    hn = jnp.maximum((t - m) * sc + be_ref[...], 0.0)
    out_ref[...] = jnp.dot(hn, w_ref[...],
                           preferred_element_type=jnp.float32) * dq


def _tc_apply(agg, hs, dq, st, b4, g4, be4, W4):
    return pl.pallas_call(
        _apply_body,
        grid=(NBLK,),
        in_specs=[
            pl.BlockSpec((BLK4, 128), lambda i: (i, 0)),
            pl.BlockSpec((BLK4, 128), lambda i: (i, 0)),
            pl.BlockSpec((BLK4, 128), lambda i: (i, 0)),
            pl.BlockSpec((8, 128), lambda i: (0, 0)),
            pl.BlockSpec((1, 128), lambda i: (0, 0)),
            pl.BlockSpec((1, 128), lambda i: (0, 0)),
            pl.BlockSpec((1, 128), lambda i: (0, 0)),
            pl.BlockSpec((128, 128), lambda i: (0, 0)),
        ],
        out_specs=pl.BlockSpec((BLK4, 128), lambda i: (i, 0)),
        out_shape=jax.ShapeDtypeStruct((NP4, 128), jnp.float32),
    )(agg, hs, dq, st, b4, g4, be4, W4)


def _pool_body(agg_ref, hs_ref, dq_ref, b_ref, bid_ref, ps_ref, pc_ref):
    i = pl.program_id(0)
    t = (agg_ref[...] + hs_ref[...]) * dq_ref[...] + b_ref[...]
    rows = lax.broadcasted_iota(jnp.int32, (BLK4, 1), 0) + i * BLK4
    t = jnp.where(rows < NROW4, t, 0.0)

    @pl.when(i == 0)
    def _():
        ps_ref[...] = jnp.zeros((B, H), jnp.float32)
        pc_ref[...] = jnp.zeros((B, 1), jnp.float32)

    lane_b = lax.broadcasted_iota(jnp.int32, (BLK4, B), 1)
    psum = jnp.zeros((B, H), jnp.float32)
    cnt = jnp.zeros((B, 1), jnp.float32)
    for q in range(4):
        oh = (bid_ref[:, q:q + 1] == lane_b).astype(jnp.float32)
        psum += lax.dot_general(oh, t[:, 32 * q:32 * q + 32],
                                (((0,), (0,)), ((), ())),
                                preferred_element_type=jnp.float32)
        cnt += jnp.sum(oh, axis=0)[:, None]
    ps_ref[...] += psum
    pc_ref[...] += cnt


def _tc_pool(agg, hs, dq, b4, bid4):
    return pl.pallas_call(
        _pool_body,
        grid=(NBLK,),
        in_specs=[
            pl.BlockSpec((BLK4, 128), lambda i: (i, 0)),
            pl.BlockSpec((BLK4, 128), lambda i: (i, 0)),
            pl.BlockSpec((BLK4, 128), lambda i: (i, 0)),
            pl.BlockSpec((1, 128), lambda i: (0, 0)),
            pl.BlockSpec((BLK4, 4), lambda i: (i, 0)),
        ],
        out_specs=[
            pl.BlockSpec((B, H), lambda i: (0, 0)),
            pl.BlockSpec((B, 1), lambda i: (0, 0)),
        ],
        out_shape=[
            jax.ShapeDtypeStruct((B, H), jnp.float32),
            jax.ShapeDtypeStruct((B, 1), jnp.float32),
        ],
    )(agg, hs, dq, b4, bid4)


def _head_body(ps_ref, pc_ref, gf_ref, gw_ref, gb_ref, pw1_ref, pb1_ref,
               pw2_ref, pb2_ref, out_ref):
    pooled = ps_ref[...] / jnp.maximum(pc_ref[...], 1.0)
    grepr = jnp.maximum(
        jnp.dot(gf_ref[...], gw_ref[...], preferred_element_type=jnp.float32)
        + gb_ref[...], 0.0)
    comb = jnp.concatenate([pooled, grepr], axis=1)
    hid = jnp.maximum(
        jnp.dot(comb, pw1_ref[...], preferred_element_type=jnp.float32)
        + pb1_ref[...], 0.0)
    out_ref[...] = (jnp.dot(hid, pw2_ref[...], preferred_element_type=jnp.float32)
                    + pb2_ref[...])


def _tc_head(ps, pc, gf, gW, gb, pW1, pb1, pW2, pb2):
    return pl.pallas_call(
        _head_body,
        out_shape=jax.ShapeDtypeStruct((B, T), jnp.float32),
    )(ps, pc, gf, gW, gb, pW1, pb1, pW2, pb2)


_F_FOLD = np.kron(np.ones((4, 4), np.float32), np.eye(32, dtype=np.float32))
_F_CNT = np.kron(np.eye(4, dtype=np.float32),
                 np.kron(np.ones((2, 2), np.float32),
                         np.eye(16, dtype=np.float32)))


def _kron4(W):
    return jnp.kron(jnp.asarray(np.eye(4, dtype=np.float32)), W)


def kernel(x, edge_index, batch, global_features, W1, b1, W2, b2, W3, b3,
           g1, be1, g2, be2, gW, gb, pW1, pb1, pW2, pb2):
    xp = jnp.pad(x, ((0, NPAD - N), (0, 0))).reshape(NP4, 128)
    srcr = jnp.pad(edge_index[0], (0, EPAD - E)).reshape(EROWS, 128)
    dstr = jnp.pad(edge_index[1], (0, EPAD - E), constant_values=N).reshape(EROWS, 128)
    bid4 = jnp.pad(batch, (0, NPAD - N), constant_values=B).reshape(NP4, 4)
    zinit = jnp.zeros((NPAD, 16), jnp.float32)
    onesrow = jnp.ones((128, 16), jnp.float32)

    f_cnt = jnp.asarray(_F_CNT)
    f_fold = jnp.asarray(_F_FOLD)
    cntp = _deg_call(dstr, zinit, onesrow).reshape(NP4, 128)
    hs1, dq = _tc_a(xp, cntp, f_cnt, _kron4(W1))

    def edge(hs):
        hs_t = jnp.transpose(hs.reshape(NPAD, 2, 16), (1, 0, 2))
        out = _edge_call(hs_t, zinit, srcr, dstr)
        return out.reshape(NP4, 128)

    def tile4(v):
        return jnp.tile(v, 4).reshape(1, 128)

    agg1 = edge(hs1)
    b1r, g1r, be1r = tile4(b1), tile4(g1), tile4(be1)
    st1 = _tc_stats(agg1, hs1, dq, b1r, f_fold)
    hs2 = _tc_apply(agg1, hs1, dq, st1, b1r, g1r, be1r, _kron4(W2))
    agg2 = edge(hs2)
    b2r, g2r, be2r = tile4(b2), tile4(g2), tile4(be2)
    st2 = _tc_stats(agg2, hs2, dq, b2r, f_fold)
    hs3 = _tc_apply(agg2, hs2, dq, st2, b2r, g2r, be2r, _kron4(W3))
    agg3 = edge(hs3)
    ps, pc = _tc_pool(agg3, hs3, dq, tile4(b3), bid4)
    return _tc_head(ps, pc, global_features, gW, gb.reshape(1, G),
                    pW1, pb1.reshape(1, H), pW2, pb2.reshape(1, T))


# speedup vs baseline: 1.3153x; 1.3153x over previous
import jax
import jax.numpy as jnp
import numpy as np
from jax import lax
from jax.experimental import pallas as pl
from jax.experimental.pallas import tpu as pltpu
from jax.experimental.pallas import tpu_sc as plsc

N = 100000
E = 1600000
D = 32
H = 32
G = 16
B = 128
T = 5

NPAD = 100352
EPAD = 1638400
EROWS = EPAD // 128
NPT = NPAD // 16
ERT = EROWS // 16
DRT = EROWS // 32

NP4 = NPAD // 4
BLK4 = 256
NBLK = NP4 // BLK4
NROW4 = N // 4

_mesh = plsc.VectorSubcoreMesh(
    core_axis_name="c", subcore_axis_name="s", num_cores=2, num_subcores=16)


def _deg_body(dstr_hbm, zinit_hbm, onesrow_hbm, out_hbm, idx_v, ones_v, acc_sh, sem):
    c = lax.axis_index("c")
    s = lax.axis_index("s")
    w = c * 16 + s
    nb = s * NPT
    pltpu.sync_copy(zinit_hbm.at[pl.ds(nb, NPT)], acc_sh.at[pl.ds(nb, NPT)])
    pltpu.sync_copy(onesrow_hbm, ones_v)
    plsc.subcore_barrier()

    def outer(i, _):
        pltpu.sync_copy(dstr_hbm.at[pl.ds(w * DRT + i * 16, 16)], idx_v)
        sd = [pltpu.async_copy(ones_v, acc_sh.at[idx_v.at[j]], sem, add=True)
              for j in range(16)]
        for d in sd:
            d.wait()
        return 0

    lax.fori_loop(0, DRT // 16, outer, 0)
    plsc.subcore_barrier()
    pltpu.sync_copy(acc_sh.at[pl.ds(nb, NPT)],
                    out_hbm.at[pl.ds(nb, NPT), c])


_deg_call = pl.kernel(
    _deg_body,
    out_type=jax.ShapeDtypeStruct((NPAD, 2, 16), jnp.float32),
    mesh=_mesh,
    scratch_types=[
        pltpu.VMEM((16, 128), jnp.int32),
        pltpu.VMEM((128, 16), jnp.float32),
        pltpu.VMEM_SHARED((NPAD, 16), jnp.float32),
        pltpu.SemaphoreType.DMA,
    ],
    compiler_params=pltpu.CompilerParams(use_tc_tiling_on_sc=False),
)


def _edge_body(hsg_hbm, zinit_hbm, srcr2_hbm, dstr_hbm, out_hbm, src_v, dst_v,
               rows_v, acc_sh, *sems):
    gsems, ssems = sems[:8], sems[8:]
    c = lax.axis_index("c")
    s = lax.axis_index("s")
    nb = s * NPT
    pltpu.sync_copy(zinit_hbm.at[pl.ds(nb, NPT)], acc_sh.at[pl.ds(nb, NPT)])
    plsc.subcore_barrier()
    er0 = s * ERT
    src_c = srcr2_hbm.at[c]

    def outer(i, _):
        pltpu.sync_copy(src_c.at[pl.ds(er0 + i * 32, 32)], src_v)
        pltpu.sync_copy(dstr_hbm.at[pl.ds(er0 + i * 32, 32)], dst_v)

        def sub(m, _):
            @pl.when(i * 4 + m > 0)
            def _():
                for k in range(8):
                    pltpu.make_async_copy(
                        rows_v.at[k], acc_sh.at[dst_v.at[m * 8 + k]],
                        ssems[k]).wait()

            gd = [pltpu.async_copy(hsg_hbm.at[src_v.at[m * 8 + k]],
                                   rows_v.at[k], gsems[k])
                  for k in range(8)]
            for k in range(8):
                gd[k].wait()
                pltpu.async_copy(rows_v.at[k], acc_sh.at[dst_v.at[m * 8 + k]],
                                 ssems[k], add=True)
            return 0

        lax.fori_loop(0, 4, sub, 0)
        return 0

    lax.fori_loop(0, ERT // 32, outer, 0)
    for k in range(8):
        pltpu.make_async_copy(rows_v.at[k], acc_sh.at[dst_v.at[31]],
                              ssems[k]).wait()
    plsc.subcore_barrier()
    pltpu.sync_copy(acc_sh.at[pl.ds(nb, NPT)],
                    out_hbm.at[pl.ds(nb, NPT), c])


_edge_call = pl.kernel(
    _edge_body,
    out_type=jax.ShapeDtypeStruct((NPAD, 2, 16), jnp.float32),
    mesh=_mesh,
    scratch_types=[
        pltpu.VMEM((32, 128), jnp.int32),
        pltpu.VMEM((32, 128), jnp.int32),
        pltpu.VMEM((8, 128, 16), jnp.float32),
        pltpu.VMEM_SHARED((NPAD, 16), jnp.float32),
    ] + [pltpu.SemaphoreType.DMA] * 16,
    compiler_params=pltpu.CompilerParams(use_tc_tiling_on_sc=False),
)


def _a_body(x_ref, cnt_ref, fc_ref, w14_ref, hs_ref, dq_ref):
    cnt = jnp.dot(jnp.maximum(cnt_ref[...], 0.0), fc_ref[...],
                  preferred_element_type=jnp.float32)
    dq = lax.rsqrt(1.0 + cnt)
    hs_ref[...] = jnp.dot(x_ref[...], w14_ref[...],
                          preferred_element_type=jnp.float32) * dq
    dq_ref[...] = dq


def _tc_a(xp, cntp, Fc, W14):
    return pl.pallas_call(
        _a_body,
        grid=(NBLK,),
        in_specs=[
            pl.BlockSpec((BLK4, 128), lambda i: (i, 0)),
            pl.BlockSpec((BLK4, 128), lambda i: (i, 0)),
            pl.BlockSpec((128, 128), lambda i: (0, 0)),
            pl.BlockSpec((128, 128), lambda i: (0, 0)),
        ],
        out_specs=[
            pl.BlockSpec((BLK4, 128), lambda i: (i, 0)),
            pl.BlockSpec((BLK4, 128), lambda i: (i, 0)),
        ],
        out_shape=[
            jax.ShapeDtypeStruct((NP4, 128), jnp.float32),
            jax.ShapeDtypeStruct((NP4, 128), jnp.float32),
        ],
    )(xp, cntp, Fc, W14)


def _layer_body(agg_ref, hs_ref, dq_ref, b_ref, g_ref, be_ref, f_ref, w_ref,
                out_ref, st_ref):
    ph = pl.program_id(0)
    i = pl.program_id(1)
    dq = dq_ref[...]
    t = (agg_ref[...] + hs_ref[...]) * dq + b_ref[...]

    @pl.when(ph == 0)
    def _():
        rows = lax.broadcasted_iota(jnp.int32, (BLK4, 1), 0) + i * BLK4
        tm = jnp.where(rows < NROW4, t, 0.0)
        s0 = jnp.sum(tm, axis=0)
        s1 = jnp.sum(tm * tm, axis=0)
        stacked = jnp.concatenate(
            [s0[None, :], s1[None, :], jnp.zeros((6, 128), jnp.float32)],
            axis=0)

        @pl.when(i == 0)
        def _():
            st_ref[...] = jnp.zeros((8, 128), jnp.float32)

        st_ref[...] += jnp.dot(stacked, f_ref[...],
                               preferred_element_type=jnp.float32)

    @pl.when(ph == 1)
    def _():
        m = st_ref[0:1, :] / N
        v = st_ref[1:2, :] / N - m * m
        sc = g_ref[...] * lax.rsqrt(v + 1e-5)
        hn = jnp.maximum((t - m) * sc + be_ref[...], 0.0)
        out_ref[...] = jnp.dot(hn, w_ref[...],
                               preferred_element_type=jnp.float32) * dq


def _tc_layer(agg, hs, dq, b4, g4, be4, F, W4):
    return pl.pallas_call(
        _layer_body,
        grid=(2, NBLK),
        in_specs=[
            pl.BlockSpec((BLK4, 128), lambda p, i: (i, 0)),
            pl.BlockSpec((BLK4, 128), lambda p, i: (i, 0)),
            pl.BlockSpec((BLK4, 128), lambda p, i: (i, 0)),
            pl.BlockSpec((1, 128), lambda p, i: (0, 0)),
            pl.BlockSpec((1, 128), lambda p, i: (0, 0)),
            pl.BlockSpec((1, 128), lambda p, i: (0, 0)),
            pl.BlockSpec((128, 128), lambda p, i: (0, 0)),
            pl.BlockSpec((128, 128), lambda p, i: (0, 0)),
        ],
        out_specs=pl.BlockSpec((BLK4, 128), lambda p, i: (i, 0)),
        out_shape=jax.ShapeDtypeStruct((NP4, 128), jnp.float32),
        scratch_shapes=[pltpu.VMEM((8, 128), jnp.float32)],
    )(agg, hs, dq, b4, g4, be4, F, W4)


def _tail_body(agg_ref, hs_ref, dq_ref, b_ref, bid_ref, gf_ref, gw_ref,
               gb_ref, pw1_ref, pb1_ref, pw2_ref, pb2_ref, out_ref,
               ps_ref, pc_ref):
    i = pl.program_id(0)

    @pl.when(i < NBLK)
    def _():
        t = (agg_ref[...] + hs_ref[...]) * dq_ref[...] + b_ref[...]
        rows = lax.broadcasted_iota(jnp.int32, (BLK4, 1), 0) + i * BLK4
        t = jnp.where(rows < NROW4, t, 0.0)

        @pl.when(i == 0)
        def _():
            ps_ref[...] = jnp.zeros((B, H), jnp.float32)
            pc_ref[...] = jnp.zeros((B, 1), jnp.float32)

        lane_b = lax.broadcasted_iota(jnp.int32, (BLK4, B), 1)
        psum = jnp.zeros((B, H), jnp.float32)
        cnt = jnp.zeros((B, 1), jnp.float32)
        for q in range(4):
            oh = (bid_ref[:, q:q + 1] == lane_b).astype(jnp.float32)
            psum += lax.dot_general(oh, t[:, 32 * q:32 * q + 32],
                                    (((0,), (0,)), ((), ())),
                                    preferred_element_type=jnp.float32)
            cnt += jnp.sum(oh, axis=0)[:, None]
        ps_ref[...] += psum
        pc_ref[...] += cnt

    @pl.when(i == NBLK)
    def _():
        pooled = ps_ref[...] / jnp.maximum(pc_ref[...], 1.0)
        grepr = jnp.maximum(
            jnp.dot(gf_ref[...], gw_ref[...],
                    preferred_element_type=jnp.float32) + gb_ref[...], 0.0)
        comb = jnp.concatenate([pooled, grepr], axis=1)
        hid = jnp.maximum(
            jnp.dot(comb, pw1_ref[...], preferred_element_type=jnp.float32)
            + pb1_ref[...], 0.0)
        out_ref[...] = (jnp.dot(hid, pw2_ref[...],
                                preferred_element_type=jnp.float32)
                        + pb2_ref[...])


def _tc_tail(agg, hs, dq, b4, bid4, gf, gW, gb, pW1, pb1, pW2, pb2):
    blk = lambda i: (jnp.minimum(i, NBLK - 1), 0)
    return pl.pallas_call(
        _tail_body,
        grid=(NBLK + 1,),
        in_specs=[
            pl.BlockSpec((BLK4, 128), blk),
            pl.BlockSpec((BLK4, 128), blk),
            pl.BlockSpec((BLK4, 128), blk),
            pl.BlockSpec((1, 128), lambda i: (0, 0)),
            pl.BlockSpec((BLK4, 4), blk),
            pl.BlockSpec((B, G), lambda i: (0, 0)),
            pl.BlockSpec((G, G), lambda i: (0, 0)),
            pl.BlockSpec((1, G), lambda i: (0, 0)),
            pl.BlockSpec((H + G, H), lambda i: (0, 0)),
            pl.BlockSpec((1, H), lambda i: (0, 0)),
            pl.BlockSpec((H, T), lambda i: (0, 0)),
            pl.BlockSpec((1, T), lambda i: (0, 0)),
        ],
        out_specs=pl.BlockSpec((B, T), lambda i: (0, 0)),
        out_shape=jax.ShapeDtypeStruct((B, T), jnp.float32),
        scratch_shapes=[
            pltpu.VMEM((B, H), jnp.float32),
            pltpu.VMEM((B, 1), jnp.float32),
        ],
    )(agg, hs, dq, b4, bid4, gf, gW, gb, pW1, pb1, pW2, pb2)


_F_FOLD = np.kron(np.ones((4, 4), np.float32), np.eye(32, dtype=np.float32))
_F_CNT = np.kron(np.eye(4, dtype=np.float32),
                 np.kron(np.ones((2, 2), np.float32),
                         np.eye(16, dtype=np.float32)))


def _kron4(W):
    return jnp.kron(jnp.asarray(np.eye(4, dtype=np.float32)), W)


def kernel(x, edge_index, batch, global_features, W1, b1, W2, b2, W3, b3,
           g1, be1, g2, be2, gW, gb, pW1, pb1, pW2, pb2):
    xp = jnp.pad(x, ((0, NPAD - N), (0, 0))).reshape(NP4, 128)
    src2 = 2 * jnp.pad(edge_index[0], (0, EPAD - E))
    srcr2 = jnp.stack([src2, src2 + 1]).reshape(2, EROWS, 128)
    dstr = jnp.pad(edge_index[1], (0, EPAD - E), constant_values=N).reshape(EROWS, 128)
    bid4 = jnp.pad(batch, (0, NPAD - N), constant_values=B).reshape(NP4, 4)
    zinit = jnp.zeros((NPAD, 16), jnp.float32)
    onesrow = jnp.ones((128, 16), jnp.float32)

    f_cnt = jnp.asarray(_F_CNT)
    f_fold = jnp.asarray(_F_FOLD)
    cntp = _deg_call(dstr, zinit, onesrow).reshape(NP4, 128)
    hs1, dq = _tc_a(xp, cntp, f_cnt, _kron4(W1))

    def edge(hs):
        out = _edge_call(hs.reshape(2 * NPAD, 16), zinit, srcr2, dstr)
        return out.reshape(NP4, 128)

    def tile4(v):
        return jnp.tile(v, 4).reshape(1, 128)

    agg1 = edge(hs1)
    hs2 = _tc_layer(agg1, hs1, dq, tile4(b1), tile4(g1), tile4(be1),
                    f_fold, _kron4(W2))
    agg2 = edge(hs2)
    hs3 = _tc_layer(agg2, hs2, dq, tile4(b2), tile4(g2), tile4(be2),
                    f_fold, _kron4(W3))
    agg3 = edge(hs3)
    return _tc_tail(agg3, hs3, dq, tile4(b3), bid4, global_features, gW,
                    gb.reshape(1, G), pW1, pb1.reshape(1, H), pW2,
                    pb2.reshape(1, T))
